# baseline (speedup 1.0000x reference)
import functools

import jax
import jax.numpy as jnp
from jax import lax
from jax.experimental import pallas as pl
from jax.experimental.pallas import tpu as pltpu
from jax.experimental.pallas import tpu_sc as plsc

N = 10000
E = 320000
D = 128
G = 64
NC = 2
NS = 16
NW = NC * NS
CW = 128
CH = 80
IH = CH // 2
E_PAD = NW * CH * CW
RPT = 632
N_ACC = NS * RPT
PN = 400
PC = 80
PJ = PN // PC

_mesh = plsc.VectorSubcoreMesh(core_axis_name="c", subcore_axis_name="s")


def _sc_deg_body(dst_hbm, zeros_hbm, ones_hbm, out_hbm, dstv, onesv, acc):
    c = lax.axis_index("c")
    s = lax.axis_index("s")
    wid = c * NS + s
    pltpu.sync_copy(ones_hbm, onesv)
    pltpu.sync_copy(zeros_hbm.at[pl.ds(s * RPT, RPT)],
                    acc.at[pl.ds(s * RPT, RPT)])
    plsc.subcore_barrier()

    for half in range(2):
        pltpu.sync_copy(dst_hbm.at[wid, pl.ds(half * IH, IH)], dstv)

        @pl.loop(0, IH)
        def _(j):
            pltpu.sync_copy(onesv, acc.at[dstv.at[j]], add=True)

    plsc.subcore_barrier()
    pltpu.sync_copy(acc.at[pl.ds(s * RPT, RPT)],
                    out_hbm.at[c, pl.ds(s * RPT, RPT)])


_sc_deg = functools.partial(
    pl.kernel,
    out_type=jax.ShapeDtypeStruct((NC, N_ACC, D), jnp.float32),
    mesh=_mesh,
    scratch_types=[
        pltpu.VMEM((IH, CW), jnp.int32),
        pltpu.VMEM((CW, D), jnp.float32),
        pltpu.VMEM_SHARED((N_ACC, D), jnp.float32),
    ],
)(_sc_deg_body)


def _sc_agg_body(y_hbm, src_hbm, dst_hbm, zeros_hbm, out_hbm,
                 srcv, dstv, buf0, buf1, acc, sem0, sem1):
    c = lax.axis_index("c")
    s = lax.axis_index("s")
    wid = c * NS + s
    pltpu.sync_copy(zeros_hbm.at[pl.ds(s * RPT, RPT)],
                    acc.at[pl.ds(s * RPT, RPT)])
    plsc.subcore_barrier()

    for half in range(2):
        pltpu.sync_copy(src_hbm.at[wid, pl.ds(half * IH, IH)], srcv)
        pltpu.sync_copy(dst_hbm.at[wid, pl.ds(half * IH, IH)], dstv)
        pltpu.async_copy(y_hbm.at[srcv.at[0]], buf0, sem0)
        pltpu.async_copy(y_hbm.at[srcv.at[1]], buf1, sem1)

        @pl.loop(0, IH, step=2)
        def _(j):
            for b, (buf, sem) in enumerate(((buf0, sem0), (buf1, sem1))):
                jj = j + b
                pltpu.make_async_copy(y_hbm.at[srcv.at[jj]], buf, sem).wait()
                pltpu.sync_copy(buf, acc.at[dstv.at[jj]], add=True)

                @pl.when(jj + 2 < IH)
                def _():
                    pltpu.async_copy(y_hbm.at[srcv.at[jj + 2]], buf, sem)

    plsc.subcore_barrier()
    pltpu.sync_copy(acc.at[pl.ds(s * RPT, RPT)],
                    out_hbm.at[c, pl.ds(s * RPT, RPT)])


_sc_agg = functools.partial(
    pl.kernel,
    out_type=jax.ShapeDtypeStruct((NC, N_ACC, D), jnp.float32),
    mesh=_mesh,
    scratch_types=[
        pltpu.VMEM((IH, CW), jnp.int32),
        pltpu.VMEM((IH, CW), jnp.int32),
        pltpu.VMEM((CW, D), jnp.float32),
        pltpu.VMEM((CW, D), jnp.float32),
        pltpu.VMEM_SHARED((N_ACC, D), jnp.float32),
        pltpu.SemaphoreType.DMA,
        pltpu.SemaphoreType.DMA,
    ],
)(_sc_agg_body)


def _sc_pool_body(h_hbm, batch_hbm, zeros_hbm, ones_hbm,
                  sum_hbm, cnt_hbm, batchv, buf, onesv, sacc, cacc):
    c = lax.axis_index("c")
    s = lax.axis_index("s")
    wid = c * NS + s
    pltpu.sync_copy(ones_hbm.at[pl.ds(0, PC)], onesv)

    @pl.when(s == 0)
    def _():
        pltpu.sync_copy(zeros_hbm.at[pl.ds(0, 72)], sacc)
        pltpu.sync_copy(zeros_hbm.at[pl.ds(72, 72)], cacc)

    plsc.subcore_barrier()

    @pl.when(wid < N // PN)
    def _():
        pltpu.sync_copy(batch_hbm.at[wid], batchv)

        @pl.loop(0, PJ)
        def _(j):
            pltpu.sync_copy(h_hbm.at[pl.ds(wid * PN + j * PC, PC)], buf)
            pltpu.sync_copy(buf, sacc.at[batchv.at[j]], add=True)
            pltpu.sync_copy(onesv, cacc.at[batchv.at[j]], add=True)

    plsc.subcore_barrier()

    @pl.when(s == 0)
    def _():
        pltpu.sync_copy(sacc, sum_hbm.at[c])
        pltpu.sync_copy(cacc, cnt_hbm.at[c])


_sc_pool = functools.partial(
    pl.kernel,
    out_type=(jax.ShapeDtypeStruct((NC, 72, D), jnp.float32),
              jax.ShapeDtypeStruct((NC, 72, D), jnp.float32)),
    mesh=_mesh,
    scratch_types=[
        pltpu.VMEM((8, PC), jnp.int32),
        pltpu.VMEM((PC, D), jnp.float32),
        pltpu.VMEM((PC, D), jnp.float32),
        pltpu.VMEM_SHARED((72, D), jnp.float32),
        pltpu.VMEM_SHARED((72, D), jnp.float32),
    ],
)(_sc_pool_body)


BLK = 1000


def _tc1_body(x_ref, w_ref, d0_ref, d1_ref, y_ref, dinv_ref):
    dinv = lax.rsqrt(d0_ref[:, :1] + d1_ref[:, :1] + 1.0)
    xw = lax.dot_general(x_ref[...], w_ref[...], (((1,), (0,)), ((), ())),
                         precision=lax.Precision.HIGHEST,
                         preferred_element_type=jnp.float32)
    y_ref[...] = xw * dinv
    dinv_ref[...] = jnp.broadcast_to(dinv, (BLK, 8))


def _tc2_body(p0_ref, p1_ref, y1_ref, dinv_ref, b_ref, w_ref, y2_ref):
    dv = dinv_ref[:, :1]
    h = jnp.maximum(dv * (p0_ref[...] + p1_ref[...] + y1_ref[...]) + b_ref[...],
                    0.0)
    y2_ref[...] = lax.dot_general(h, w_ref[...], (((1,), (0,)), ((), ())),
                                  precision=lax.Precision.HIGHEST,
                                  preferred_element_type=jnp.float32) * dv


def _tc3_body(p0_ref, p1_ref, y2_ref, dinv_ref, b_ref, h_ref):
    dv = dinv_ref[:, :1]
    h_ref[...] = jnp.maximum(
        dv * (p0_ref[...] + p1_ref[...] + y2_ref[...]) + b_ref[...], 0.0)


def _tc4_body(s0_ref, s1_ref, c0_ref, c1_ref, out_ref):
    ssum = s0_ref[...] + s1_ref[...]
    cnt = c0_ref[:, :1] + c1_ref[:, :1]
    out_ref[...] = (ssum / jnp.maximum(cnt, 1.0))[:G, :]


def _row_spec(w):
    return pl.BlockSpec((BLK, w), lambda i: (i, 0))


def _full_spec(shape):
    return pl.BlockSpec(shape, lambda i: (0, 0))


def kernel(x, edge_index, batch, W1, b1, W2, b2):
    src = edge_index[0].astype(jnp.int32)
    dst = edge_index[1].astype(jnp.int32)
    npad = E_PAD - E
    src3 = jnp.concatenate([src, jnp.zeros((npad,), jnp.int32)]).reshape(NW, CH, CW)
    dst3 = jnp.concatenate([dst, jnp.full((npad,), N, jnp.int32)]).reshape(NW, CH, CW)
    batch3 = jnp.zeros((N // PN, 8, PC), jnp.int32)
    batch3 = batch3.at[:, :PJ, :].set(batch.astype(jnp.int32).reshape(N // PN, PJ, PC))
    zeros128 = jnp.zeros((N_ACC, D), jnp.float32)
    ones128 = jnp.ones((CW, D), jnp.float32)
    b1r = b1.reshape(1, D)
    b2r = b2.reshape(1, D)

    deg = _sc_deg(dst3, zeros128, ones128)
    d0 = deg[0, :N]
    d1 = deg[1, :N]

    grid = N // BLK
    y1, dinv = pl.pallas_call(
        _tc1_body,
        grid=(grid,),
        in_specs=[_row_spec(D), _full_spec((D, D)), _row_spec(D), _row_spec(D)],
        out_specs=[_row_spec(D), _row_spec(8)],
        out_shape=[jax.ShapeDtypeStruct((N, D), jnp.float32),
                   jax.ShapeDtypeStruct((N, 8), jnp.float32)],
    )(x, W1, d0, d1)

    agg1 = _sc_agg(y1, src3, dst3, zeros128)

    y2 = pl.pallas_call(
        _tc2_body,
        grid=(grid,),
        in_specs=[_row_spec(D), _row_spec(D), _row_spec(D), _row_spec(8),
                  _full_spec((1, D)), _full_spec((D, D))],
        out_specs=_row_spec(D),
        out_shape=jax.ShapeDtypeStruct((N, D), jnp.float32),
    )(agg1[0, :N], agg1[1, :N], y1, dinv, b1r, W2)

    agg2 = _sc_agg(y2, src3, dst3, zeros128)

    h2 = pl.pallas_call(
        _tc3_body,
        grid=(grid,),
        in_specs=[_row_spec(D), _row_spec(D), _row_spec(D), _row_spec(8),
                  _full_spec((1, D))],
        out_specs=_row_spec(D),
        out_shape=jax.ShapeDtypeStruct((N, D), jnp.float32),
    )(agg2[0, :N], agg2[1, :N], y2, dinv, b2r)

    sums, cnts = _sc_pool(h2, batch3, zeros128, ones128)

    graph_emb = pl.pallas_call(
        _tc4_body,
        grid=(1,),
        in_specs=[_full_spec((72, D)), _full_spec((72, D)),
                  _full_spec((72, D)), _full_spec((72, D))],
        out_specs=_full_spec((G, D)),
        out_shape=jax.ShapeDtypeStruct((G, D), jnp.float32),
    )(sums[0], sums[1], cnts[0], cnts[1])

    return (h2, graph_emb)


# speedup vs baseline: 2.7241x; 2.7241x over previous
import functools

import jax
import jax.numpy as jnp
from jax import lax
from jax.experimental import pallas as pl
from jax.experimental.pallas import tpu as pltpu
from jax.experimental.pallas import tpu_sc as plsc

N = 10000
E = 320000
D = 128
G = 64
NC = 2
NS = 16
NW = NC * NS
CW = 128
CH = 80
IH = CH // 2
E_PAD = NW * CH * CW
RPT = 632
N_ACC = NS * RPT
PN = 400
PC = 80
PJ = PN // PC

_mesh = plsc.VectorSubcoreMesh(core_axis_name="c", subcore_axis_name="s")


def _sc_deg_body(dst_hbm, zeros_hbm, ones_hbm, out_hbm, dstv, onesv, acc):
    c = lax.axis_index("c")
    s = lax.axis_index("s")
    wid = c * NS + s
    pltpu.sync_copy(ones_hbm, onesv)
    pltpu.sync_copy(zeros_hbm.at[pl.ds(s * RPT, RPT)],
                    acc.at[pl.ds(s * RPT, RPT)])
    plsc.subcore_barrier()

    for half in range(2):
        pltpu.sync_copy(dst_hbm.at[wid, pl.ds(half * IH, IH)], dstv)

        @pl.loop(0, IH)
        def _(j):
            pltpu.sync_copy(onesv, acc.at[dstv.at[j]], add=True)

    plsc.subcore_barrier()
    pltpu.sync_copy(acc.at[pl.ds(s * RPT, RPT)],
                    out_hbm.at[c, pl.ds(s * RPT, RPT)])


_sc_deg = functools.partial(
    pl.kernel,
    out_type=jax.ShapeDtypeStruct((NC, N_ACC, D), jnp.float32),
    mesh=_mesh,
    scratch_types=[
        pltpu.VMEM((IH, CW), jnp.int32),
        pltpu.VMEM((CW, D), jnp.float32),
        pltpu.VMEM_SHARED((N_ACC, D), jnp.float32),
    ],
)(_sc_deg_body)


def _sc_agg_body(y_hbm, src_hbm, dst_hbm, zeros_hbm, out_hbm,
                 srcv, dstv, buf0, buf1, acc, sem0, sem1):
    c = lax.axis_index("c")
    s = lax.axis_index("s")
    wid = c * NS + s
    pltpu.sync_copy(zeros_hbm.at[pl.ds(s * RPT, RPT)],
                    acc.at[pl.ds(s * RPT, RPT)])
    plsc.subcore_barrier()

    for half in range(2):
        pltpu.sync_copy(src_hbm.at[wid, pl.ds(half * IH, IH)], srcv)
        pltpu.sync_copy(dst_hbm.at[wid, pl.ds(half * IH, IH)], dstv)
        pltpu.async_copy(y_hbm.at[srcv.at[0]], buf0, sem0)
        pltpu.async_copy(y_hbm.at[srcv.at[1]], buf1, sem1)

        @pl.loop(0, IH, step=2)
        def _(j):
            for b, (buf, sem) in enumerate(((buf0, sem0), (buf1, sem1))):
                jj = j + b
                pltpu.make_async_copy(y_hbm.at[srcv.at[jj]], buf, sem).wait()
                pltpu.sync_copy(buf, acc.at[dstv.at[jj]], add=True)

                @pl.when(jj + 2 < IH)
                def _():
                    pltpu.async_copy(y_hbm.at[srcv.at[jj + 2]], buf, sem)

    plsc.subcore_barrier()
    pltpu.sync_copy(acc.at[pl.ds(s * RPT, RPT)],
                    out_hbm.at[c, pl.ds(s * RPT, RPT)])


_sc_agg = functools.partial(
    pl.kernel,
    out_type=jax.ShapeDtypeStruct((NC, N_ACC, D), jnp.float32),
    mesh=_mesh,
    scratch_types=[
        pltpu.VMEM((IH, CW), jnp.int32),
        pltpu.VMEM((IH, CW), jnp.int32),
        pltpu.VMEM((CW, D), jnp.float32),
        pltpu.VMEM((CW, D), jnp.float32),
        pltpu.VMEM_SHARED((N_ACC, D), jnp.float32),
        pltpu.SemaphoreType.DMA,
        pltpu.SemaphoreType.DMA,
    ],
)(_sc_agg_body)


def _sc_pool_body(h_hbm, batch_hbm, zeros_hbm, ones_hbm,
                  sum_hbm, cnt_hbm, batchv, buf, onesv, sacc, cacc):
    c = lax.axis_index("c")
    s = lax.axis_index("s")
    wid = c * NS + s
    pltpu.sync_copy(ones_hbm.at[pl.ds(0, PC)], onesv)

    @pl.when(s == 0)
    def _():
        pltpu.sync_copy(zeros_hbm.at[pl.ds(0, 72)], sacc)
        pltpu.sync_copy(zeros_hbm.at[pl.ds(72, 72)], cacc)

    plsc.subcore_barrier()

    @pl.when(wid < N // PN)
    def _():
        pltpu.sync_copy(batch_hbm.at[wid], batchv)

        @pl.loop(0, PJ)
        def _(j):
            pltpu.sync_copy(h_hbm.at[pl.ds(wid * PN + j * PC, PC)], buf)
            pltpu.sync_copy(buf, sacc.at[batchv.at[j]], add=True)
            pltpu.sync_copy(onesv, cacc.at[batchv.at[j]], add=True)

    plsc.subcore_barrier()

    @pl.when(s == 0)
    def _():
        pltpu.sync_copy(sacc, sum_hbm.at[c])
        pltpu.sync_copy(cacc, cnt_hbm.at[c])


_sc_pool = functools.partial(
    pl.kernel,
    out_type=(jax.ShapeDtypeStruct((NC, 72, D), jnp.float32),
              jax.ShapeDtypeStruct((NC, 72, D), jnp.float32)),
    mesh=_mesh,
    scratch_types=[
        pltpu.VMEM((8, PC), jnp.int32),
        pltpu.VMEM((PC, D), jnp.float32),
        pltpu.VMEM((PC, D), jnp.float32),
        pltpu.VMEM_SHARED((72, D), jnp.float32),
        pltpu.VMEM_SHARED((72, D), jnp.float32),
    ],
)(_sc_pool_body)


BLK = 1000


def _tc1_body(x_ref, w_ref, d0_ref, d1_ref, y_ref, dinv_ref):
    dinv = lax.rsqrt(d0_ref[:, :1] + d1_ref[:, :1] + 1.0)
    xw = lax.dot_general(x_ref[...], w_ref[...], (((1,), (0,)), ((), ())),
                         precision=lax.Precision.HIGHEST,
                         preferred_element_type=jnp.float32)
    y_ref[...] = xw * dinv
    dinv_ref[...] = jnp.broadcast_to(dinv, (BLK, 8))


def _tc2_body(p0_ref, p1_ref, y1_ref, dinv_ref, b_ref, w_ref, y2_ref):
    dv = dinv_ref[:, :1]
    h = jnp.maximum(dv * (p0_ref[...] + p1_ref[...] + y1_ref[...]) + b_ref[...],
                    0.0)
    y2_ref[...] = lax.dot_general(h, w_ref[...], (((1,), (0,)), ((), ())),
                                  precision=lax.Precision.HIGHEST,
                                  preferred_element_type=jnp.float32) * dv


def _tc3_body(p0_ref, p1_ref, y2_ref, dinv_ref, b_ref, h_ref):
    dv = dinv_ref[:, :1]
    h_ref[...] = jnp.maximum(
        dv * (p0_ref[...] + p1_ref[...] + y2_ref[...]) + b_ref[...], 0.0)


def _tc4_body(s0_ref, s1_ref, c0_ref, c1_ref, out_ref):
    ssum = s0_ref[...] + s1_ref[...]
    cnt = c0_ref[:, :1] + c1_ref[:, :1]
    out_ref[...] = (ssum / jnp.maximum(cnt, 1.0))[:G, :]


def _row_spec(w):
    return pl.BlockSpec((BLK, w), lambda i: (i, 0))


def _full_spec(shape):
    return pl.BlockSpec(shape, lambda i: (0, 0))


def kernel(x, edge_index, batch, W1, b1, W2, b2):
    src = edge_index[0].astype(jnp.int32)
    dst = edge_index[1].astype(jnp.int32)
    npad = E_PAD - E
    pad_src = jnp.arange(npad, dtype=jnp.int32) % N
    pad_dst = N + (jnp.arange(npad, dtype=jnp.int32) % (N_ACC - N))
    src3 = jnp.concatenate([src, pad_src]).reshape(NW, CH, CW)
    dst3 = jnp.concatenate([dst, pad_dst]).reshape(NW, CH, CW)
    batch3 = jnp.zeros((N // PN, 8, PC), jnp.int32)
    batch3 = batch3.at[:, :PJ, :].set(batch.astype(jnp.int32).reshape(N // PN, PJ, PC))
    zeros128 = jnp.zeros((N_ACC, D), jnp.float32)
    ones128 = jnp.ones((CW, D), jnp.float32)
    b1r = b1.reshape(1, D)
    b2r = b2.reshape(1, D)

    deg = _sc_deg(dst3, zeros128, ones128)
    d0 = deg[0, :N]
    d1 = deg[1, :N]

    grid = N // BLK
    y1, dinv = pl.pallas_call(
        _tc1_body,
        grid=(grid,),
        in_specs=[_row_spec(D), _full_spec((D, D)), _row_spec(D), _row_spec(D)],
        out_specs=[_row_spec(D), _row_spec(8)],
        out_shape=[jax.ShapeDtypeStruct((N, D), jnp.float32),
                   jax.ShapeDtypeStruct((N, 8), jnp.float32)],
    )(x, W1, d0, d1)

    agg1 = _sc_agg(y1, src3, dst3, zeros128)

    y2 = pl.pallas_call(
        _tc2_body,
        grid=(grid,),
        in_specs=[_row_spec(D), _row_spec(D), _row_spec(D), _row_spec(8),
                  _full_spec((1, D)), _full_spec((D, D))],
        out_specs=_row_spec(D),
        out_shape=jax.ShapeDtypeStruct((N, D), jnp.float32),
    )(agg1[0, :N], agg1[1, :N], y1, dinv, b1r, W2)

    agg2 = _sc_agg(y2, src3, dst3, zeros128)

    h2 = pl.pallas_call(
        _tc3_body,
        grid=(grid,),
        in_specs=[_row_spec(D), _row_spec(D), _row_spec(D), _row_spec(8),
                  _full_spec((1, D))],
        out_specs=_row_spec(D),
        out_shape=jax.ShapeDtypeStruct((N, D), jnp.float32),
    )(agg2[0, :N], agg2[1, :N], y2, dinv, b2r)

    sums, cnts = _sc_pool(h2, batch3, zeros128, ones128)

    graph_emb = pl.pallas_call(
        _tc4_body,
        grid=(1,),
        in_specs=[_full_spec((72, D)), _full_spec((72, D)),
                  _full_spec((72, D)), _full_spec((72, D))],
        out_specs=_full_spec((G, D)),
        out_shape=jax.ShapeDtypeStruct((G, D), jnp.float32),
    )(sums[0], sums[1], cnts[0], cnts[1])

    return (h2, graph_emb)


# speedup vs baseline: 2.8269x; 1.0378x over previous
import functools

import jax
import jax.numpy as jnp
from jax import lax
from jax.experimental import pallas as pl
from jax.experimental.pallas import tpu as pltpu
from jax.experimental.pallas import tpu_sc as plsc

N = 10000
E = 320000
D = 128
G = 64
NC = 2
NS = 16
NW = NC * NS
CW = 128
CH = 80
IH = CH // 2
E_PAD = NW * CH * CW
RPT = 632
N_ACC = NS * RPT
PN = 400
PC = 80
PJ = PN // PC

_mesh = plsc.VectorSubcoreMesh(core_axis_name="c", subcore_axis_name="s")


def _sc_deg_body(dst_hbm, zeros_hbm, ones_hbm, out_hbm, dstv, onesv, acc):
    c = lax.axis_index("c")
    s = lax.axis_index("s")
    wid = c * NS + s
    pltpu.sync_copy(ones_hbm, onesv)
    pltpu.sync_copy(zeros_hbm.at[pl.ds(s * RPT, RPT)],
                    acc.at[pl.ds(s * RPT, RPT)])
    plsc.subcore_barrier()

    for half in range(2):
        pltpu.sync_copy(dst_hbm.at[wid, pl.ds(half * IH, IH)], dstv)

        @pl.loop(0, IH)
        def _(j):
            pltpu.sync_copy(onesv, acc.at[dstv.at[j]], add=True)

    plsc.subcore_barrier()
    pltpu.sync_copy(acc.at[pl.ds(s * RPT, RPT)],
                    out_hbm.at[c, pl.ds(s * RPT, RPT)])


_sc_deg = functools.partial(
    pl.kernel,
    out_type=jax.ShapeDtypeStruct((NC, N_ACC, D), jnp.float32),
    mesh=_mesh,
    scratch_types=[
        pltpu.VMEM((IH, CW), jnp.int32),
        pltpu.VMEM((CW, D), jnp.float32),
        pltpu.VMEM_SHARED((N_ACC, D), jnp.float32),
    ],
)(_sc_deg_body)


def _sc_agg_body(y_hbm, src_hbm, dst_hbm, zeros_hbm, out_hbm,
                 srcv, dstv, buf0, buf1, acc, sem0, sem1):
    c = lax.axis_index("c")
    s = lax.axis_index("s")
    wid = c * NS + s
    pltpu.sync_copy(zeros_hbm.at[pl.ds(s * RPT, RPT)],
                    acc.at[pl.ds(s * RPT, RPT)])
    plsc.subcore_barrier()

    for half in range(2):
        pltpu.sync_copy(src_hbm.at[wid, pl.ds(half * IH, IH)], srcv)
        pltpu.sync_copy(dst_hbm.at[wid, pl.ds(half * IH, IH)], dstv)
        pltpu.async_copy(y_hbm.at[srcv.at[0]], buf0, sem0)
        pltpu.async_copy(y_hbm.at[srcv.at[1]], buf1, sem1)

        @pl.loop(0, IH, step=2)
        def _(j):
            for b, (buf, sem) in enumerate(((buf0, sem0), (buf1, sem1))):
                jj = j + b
                pltpu.make_async_copy(y_hbm.at[srcv.at[jj]], buf, sem).wait()
                pltpu.sync_copy(buf, acc.at[dstv.at[jj]], add=True)

                @pl.when(jj + 2 < IH)
                def _():
                    pltpu.async_copy(y_hbm.at[srcv.at[jj + 2]], buf, sem)

    plsc.subcore_barrier()
    pltpu.sync_copy(acc.at[pl.ds(s * RPT, RPT)],
                    out_hbm.at[c, pl.ds(s * RPT, RPT)])


_sc_agg = functools.partial(
    pl.kernel,
    out_type=jax.ShapeDtypeStruct((NC, N_ACC, D), jnp.float32),
    mesh=_mesh,
    scratch_types=[
        pltpu.VMEM((IH, CW), jnp.int32),
        pltpu.VMEM((IH, CW), jnp.int32),
        pltpu.VMEM((CW, D), jnp.float32),
        pltpu.VMEM((CW, D), jnp.float32),
        pltpu.VMEM_SHARED((N_ACC, D), jnp.float32),
        pltpu.SemaphoreType.DMA,
        pltpu.SemaphoreType.DMA,
    ],
)(_sc_agg_body)


BLK = 1000


def _tc1_body(x_ref, w_ref, d0_ref, d1_ref, y_ref, dinv_ref):
    dinv = lax.rsqrt(d0_ref[:, :1] + d1_ref[:, :1] + 1.0)
    xw = lax.dot_general(x_ref[...], w_ref[...], (((1,), (0,)), ((), ())),
                         precision=lax.Precision.HIGHEST,
                         preferred_element_type=jnp.float32)
    y_ref[...] = xw * dinv
    dinv_ref[...] = jnp.broadcast_to(dinv, (BLK, 8))


def _tc2_body(p0_ref, p1_ref, y1_ref, dinv_ref, b_ref, w_ref, y2_ref):
    dv = dinv_ref[:, :1]
    h = jnp.maximum(dv * (p0_ref[...] + p1_ref[...] + y1_ref[...]) + b_ref[...],
                    0.0)
    y2_ref[...] = lax.dot_general(h, w_ref[...], (((1,), (0,)), ((), ())),
                                  precision=lax.Precision.HIGHEST,
                                  preferred_element_type=jnp.float32) * dv


def _tc3_body(p0_ref, p1_ref, y2_ref, dinv_ref, b_ref, batch_ref,
              h_ref, ge_ref, acc_ref, cnt_ref):
    i = pl.program_id(0)
    dv = dinv_ref[:, :1]
    h = jnp.maximum(
        dv * (p0_ref[...] + p1_ref[...] + y2_ref[...]) + b_ref[...], 0.0)
    h_ref[...] = h

    onehot = (batch_ref[...] == lax.broadcasted_iota(
        jnp.int32, (BLK, G), 1)).astype(jnp.float32)
    psum = lax.dot_general(onehot, h, (((0,), (0,)), ((), ())),
                           precision=lax.Precision.HIGHEST,
                           preferred_element_type=jnp.float32)
    cnts = jnp.sum(onehot, axis=0)[:, None]

    @pl.when(i == 0)
    def _():
        acc_ref[...] = jnp.zeros((G, D), jnp.float32)
        cnt_ref[...] = jnp.zeros((G, 8), jnp.float32)

    acc_ref[...] += psum
    cnt_ref[...] += jnp.broadcast_to(cnts, (G, 8))
    ge_ref[...] = acc_ref[...] / jnp.maximum(cnt_ref[:, :1], 1.0)


def _row_spec(w):
    return pl.BlockSpec((BLK, w), lambda i: (i, 0))


def _full_spec(shape):
    return pl.BlockSpec(shape, lambda i: (0, 0))


def kernel(x, edge_index, batch, W1, b1, W2, b2):
    src = edge_index[0].astype(jnp.int32)
    dst = edge_index[1].astype(jnp.int32)
    npad = E_PAD - E
    pad_src = jnp.arange(npad, dtype=jnp.int32) % N
    pad_dst = N + (jnp.arange(npad, dtype=jnp.int32) % (N_ACC - N))
    src3 = jnp.concatenate([src, pad_src]).reshape(NW, CH, CW)
    dst3 = jnp.concatenate([dst, pad_dst]).reshape(NW, CH, CW)
    batchf = batch.astype(jnp.int32).reshape(N, 1)
    zeros128 = jnp.zeros((N_ACC, D), jnp.float32)
    ones128 = jnp.ones((CW, D), jnp.float32)
    b1r = b1.reshape(1, D)
    b2r = b2.reshape(1, D)

    deg = _sc_deg(dst3, zeros128, ones128)
    d0 = deg[0, :N]
    d1 = deg[1, :N]

    grid = N // BLK
    y1, dinv = pl.pallas_call(
        _tc1_body,
        grid=(grid,),
        in_specs=[_row_spec(D), _full_spec((D, D)), _row_spec(D), _row_spec(D)],
        out_specs=[_row_spec(D), _row_spec(8)],
        out_shape=[jax.ShapeDtypeStruct((N, D), jnp.float32),
                   jax.ShapeDtypeStruct((N, 8), jnp.float32)],
    )(x, W1, d0, d1)

    agg1 = _sc_agg(y1, src3, dst3, zeros128)

    y2 = pl.pallas_call(
        _tc2_body,
        grid=(grid,),
        in_specs=[_row_spec(D), _row_spec(D), _row_spec(D), _row_spec(8),
                  _full_spec((1, D)), _full_spec((D, D))],
        out_specs=_row_spec(D),
        out_shape=jax.ShapeDtypeStruct((N, D), jnp.float32),
    )(agg1[0, :N], agg1[1, :N], y1, dinv, b1r, W2)

    agg2 = _sc_agg(y2, src3, dst3, zeros128)

    h2, graph_emb = pl.pallas_call(
        _tc3_body,
        grid=(grid,),
        in_specs=[_row_spec(D), _row_spec(D), _row_spec(D), _row_spec(8),
                  _full_spec((1, D)), _row_spec(1)],
        out_specs=[_row_spec(D), _full_spec((G, D))],
        out_shape=[jax.ShapeDtypeStruct((N, D), jnp.float32),
                   jax.ShapeDtypeStruct((G, D), jnp.float32)],
        scratch_shapes=[pltpu.VMEM((G, D), jnp.float32),
                        pltpu.VMEM((G, 8), jnp.float32)],
    )(agg2[0, :N], agg2[1, :N], y2, dinv, b2r, batchf)

    return (h2, graph_emb)


# speedup vs baseline: 2.8536x; 1.0095x over previous
import functools

import jax
import jax.numpy as jnp
from jax import lax
from jax.experimental import pallas as pl
from jax.experimental.pallas import tpu as pltpu
from jax.experimental.pallas import tpu_sc as plsc

N = 10000
E = 320000
D = 128
G = 64
NC = 2
NS = 16
NW = NC * NS
CW = 128
CH = 80
IH = CH // 2
E_PAD = NW * CH * CW
RPT = 632
N_ACC = NS * RPT
PN = 400
PC = 80
PJ = PN // PC

_mesh = plsc.VectorSubcoreMesh(core_axis_name="c", subcore_axis_name="s")


def _sc_deg_body(dst_hbm, zeros_hbm, ones_hbm, out_hbm, dstv, onesv, acc):
    c = lax.axis_index("c")
    s = lax.axis_index("s")
    wid = c * NS + s
    pltpu.sync_copy(ones_hbm, onesv)
    pltpu.sync_copy(zeros_hbm.at[pl.ds(s * RPT, RPT)],
                    acc.at[pl.ds(s * RPT, RPT)])
    plsc.subcore_barrier()

    for half in range(2):
        pltpu.sync_copy(dst_hbm.at[wid, pl.ds(half * IH, IH)], dstv)

        @pl.loop(0, IH)
        def _(j):
            pltpu.sync_copy(onesv, acc.at[dstv.at[j]], add=True)

    plsc.subcore_barrier()
    pltpu.sync_copy(acc.at[pl.ds(s * RPT, RPT)],
                    out_hbm.at[c, pl.ds(s * RPT, RPT)])


_sc_deg = functools.partial(
    pl.kernel,
    out_type=jax.ShapeDtypeStruct((NC, N_ACC, D), jnp.float32),
    mesh=_mesh,
    scratch_types=[
        pltpu.VMEM((IH, CW), jnp.int32),
        pltpu.VMEM((CW, D), jnp.float32),
        pltpu.VMEM_SHARED((N_ACC, D), jnp.float32),
    ],
)(_sc_deg_body)


def _sc_agg_body(y_hbm, src_hbm, dst_hbm, zeros_hbm, out_hbm,
                 srcv, dstv, buf0, buf1, acc, sem0, sem1):
    c = lax.axis_index("c")
    s = lax.axis_index("s")
    wid = c * NS + s
    pltpu.sync_copy(zeros_hbm.at[pl.ds(s * RPT, RPT)],
                    acc.at[pl.ds(s * RPT, RPT)])
    plsc.subcore_barrier()

    for half in range(2):
        pltpu.sync_copy(src_hbm.at[wid, pl.ds(half * IH, IH)], srcv)
        pltpu.sync_copy(dst_hbm.at[wid, pl.ds(half * IH, IH)], dstv)
        pltpu.async_copy(y_hbm.at[srcv.at[0]], buf0, sem0)
        pltpu.async_copy(y_hbm.at[srcv.at[1]], buf1, sem1)

        @pl.loop(0, IH, step=2)
        def _(j):
            for b, (buf, sem) in enumerate(((buf0, sem0), (buf1, sem1))):
                jj = j + b
                pltpu.make_async_copy(y_hbm.at[srcv.at[jj]], buf, sem).wait()
                pltpu.sync_copy(buf, acc.at[dstv.at[jj]], add=True)

                @pl.when(jj + 2 < IH)
                def _():
                    pltpu.async_copy(y_hbm.at[srcv.at[jj + 2]], buf, sem)

    plsc.subcore_barrier()
    pltpu.sync_copy(acc.at[pl.ds(s * RPT, RPT)],
                    out_hbm.at[c, pl.ds(s * RPT, RPT)])


_sc_agg = functools.partial(
    pl.kernel,
    out_type=jax.ShapeDtypeStruct((NC, N_ACC, D), jnp.float32),
    mesh=_mesh,
    scratch_types=[
        pltpu.VMEM((IH, CW), jnp.int32),
        pltpu.VMEM((IH, CW), jnp.int32),
        pltpu.VMEM((CW, D), jnp.float32),
        pltpu.VMEM((CW, D), jnp.float32),
        pltpu.VMEM_SHARED((N_ACC, D), jnp.float32),
        pltpu.SemaphoreType.DMA,
        pltpu.SemaphoreType.DMA,
    ],
)(_sc_agg_body)


BLK = 1000


def _tc1a_body(x_ref, w_ref, xw_ref):
    xw_ref[...] = lax.dot_general(x_ref[...], w_ref[...],
                                  (((1,), (0,)), ((), ())),
                                  precision=lax.Precision.HIGHEST,
                                  preferred_element_type=jnp.float32)


def _tc1b_body(xw_ref, d0_ref, d1_ref, y_ref, dinv_ref):
    dinv = lax.rsqrt(d0_ref[:, :1] + d1_ref[:, :1] + 1.0)
    y_ref[...] = xw_ref[...] * dinv
    dinv_ref[...] = jnp.broadcast_to(dinv, (BLK, 8))


def _tc2_body(p0_ref, p1_ref, y1_ref, dinv_ref, b_ref, w_ref, y2_ref):
    dv = dinv_ref[:, :1]
    h = jnp.maximum(dv * (p0_ref[...] + p1_ref[...] + y1_ref[...]) + b_ref[...],
                    0.0)
    y2_ref[...] = lax.dot_general(h, w_ref[...], (((1,), (0,)), ((), ())),
                                  precision=lax.Precision.HIGHEST,
                                  preferred_element_type=jnp.float32) * dv


def _tc3_body(p0_ref, p1_ref, y2_ref, dinv_ref, b_ref, batch_ref,
              h_ref, ge_ref, acc_ref, cnt_ref):
    i = pl.program_id(0)
    dv = dinv_ref[:, :1]
    h = jnp.maximum(
        dv * (p0_ref[...] + p1_ref[...] + y2_ref[...]) + b_ref[...], 0.0)
    h_ref[...] = h

    onehot = (batch_ref[...] == lax.broadcasted_iota(
        jnp.int32, (BLK, G), 1)).astype(jnp.float32)
    psum = lax.dot_general(onehot, h, (((0,), (0,)), ((), ())),
                           precision=lax.Precision.HIGHEST,
                           preferred_element_type=jnp.float32)
    cnts = jnp.sum(onehot, axis=0)[:, None]

    @pl.when(i == 0)
    def _():
        acc_ref[...] = jnp.zeros((G, D), jnp.float32)
        cnt_ref[...] = jnp.zeros((G, 8), jnp.float32)

    acc_ref[...] += psum
    cnt_ref[...] += jnp.broadcast_to(cnts, (G, 8))
    ge_ref[...] = acc_ref[...] / jnp.maximum(cnt_ref[:, :1], 1.0)


def _row_spec(w):
    return pl.BlockSpec((BLK, w), lambda i: (i, 0))


def _full_spec(shape):
    return pl.BlockSpec(shape, lambda i: (0, 0))


def kernel(x, edge_index, batch, W1, b1, W2, b2):
    src = edge_index[0].astype(jnp.int32)
    dst = edge_index[1].astype(jnp.int32)
    npad = E_PAD - E
    pad_src = jnp.arange(npad, dtype=jnp.int32) % N
    pad_dst = N + (jnp.arange(npad, dtype=jnp.int32) % (N_ACC - N))
    src3 = jnp.concatenate([src, pad_src]).reshape(NW, CH, CW)
    dst3 = jnp.concatenate([dst, pad_dst]).reshape(NW, CH, CW)
    batchf = batch.astype(jnp.int32).reshape(N, 1)
    zeros128 = jnp.zeros((N_ACC, D), jnp.float32)
    ones128 = jnp.ones((CW, D), jnp.float32)
    b1r = b1.reshape(1, D)
    b2r = b2.reshape(1, D)

    deg = _sc_deg(dst3, zeros128, ones128)
    d0 = deg[0, :N]
    d1 = deg[1, :N]

    grid = N // BLK
    xw = pl.pallas_call(
        _tc1a_body,
        grid=(grid,),
        in_specs=[_row_spec(D), _full_spec((D, D))],
        out_specs=_row_spec(D),
        out_shape=jax.ShapeDtypeStruct((N, D), jnp.float32),
    )(x, W1)

    y1, dinv = pl.pallas_call(
        _tc1b_body,
        grid=(grid,),
        in_specs=[_row_spec(D), _row_spec(D), _row_spec(D)],
        out_specs=[_row_spec(D), _row_spec(8)],
        out_shape=[jax.ShapeDtypeStruct((N, D), jnp.float32),
                   jax.ShapeDtypeStruct((N, 8), jnp.float32)],
    )(xw, d0, d1)

    agg1 = _sc_agg(y1, src3, dst3, zeros128)

    y2 = pl.pallas_call(
        _tc2_body,
        grid=(grid,),
        in_specs=[_row_spec(D), _row_spec(D), _row_spec(D), _row_spec(8),
                  _full_spec((1, D)), _full_spec((D, D))],
        out_specs=_row_spec(D),
        out_shape=jax.ShapeDtypeStruct((N, D), jnp.float32),
    )(agg1[0, :N], agg1[1, :N], y1, dinv, b1r, W2)

    agg2 = _sc_agg(y2, src3, dst3, zeros128)

    h2, graph_emb = pl.pallas_call(
        _tc3_body,
        grid=(grid,),
        in_specs=[_row_spec(D), _row_spec(D), _row_spec(D), _row_spec(8),
                  _full_spec((1, D)), _row_spec(1)],
        out_specs=[_row_spec(D), _full_spec((G, D))],
        out_shape=[jax.ShapeDtypeStruct((N, D), jnp.float32),
                   jax.ShapeDtypeStruct((G, D), jnp.float32)],
        scratch_shapes=[pltpu.VMEM((G, D), jnp.float32),
                        pltpu.VMEM((G, 8), jnp.float32)],
    )(agg2[0, :N], agg2[1, :N], y2, dinv, b2r, batchf)

    return (h2, graph_emb)
